# Initial kernel scaffold; baseline (speedup 1.0000x reference)
import jax
import jax.numpy as jnp
from jax.experimental import pallas as pl


def kernel(x, edge_index, W1, b1, W2, b2):
    raise NotImplementedError("write your pallas kernel here")


# speedup vs baseline: 24.5840x; 24.5840x over previous
import functools

import jax
import jax.numpy as jnp
from jax import lax
from jax.experimental import pallas as pl
from jax.experimental.pallas import tpu as pltpu
from jax.experimental.pallas import tpu_sc as plsc

N = 10000
D = 128
E = 320000

NT = 32
NP = 10240
RSLICE = NP // 16
KSLICE = RSLICE // 128
EP = 327680
EROWS = EP // 128
RPT = EROWS // NT
SUP = 8
NSUP = RPT // SUP

BR = 2560

_mesh = plsc.VectorSubcoreMesh(core_axis_name="c", subcore_axis_name="s")


def _fill_iota_rows(zidx, base_r):
    for k in range(KSLICE):
        @pl.loop(0, 8)
        def _(t, _k=k):
            zidx[_k, pl.ds(t * 16, 16)] = (
                lax.iota(jnp.int32, 16) + (base_r + _k * 128 + t * 16))


@functools.partial(
    pl.kernel,
    out_type=jax.ShapeDtypeStruct((2, NP, 16), jnp.float32),
    mesh=_mesh,
    scratch_types=[
        pltpu.VMEM((SUP, 128), jnp.int32),
        pltpu.VMEM((128, 16), jnp.float32),
        pltpu.VMEM((KSLICE, 128), jnp.int32),
        pltpu.VMEM_SHARED((NP, 16), jnp.float32),
    ],
)
def _deg_kernel(dst_hbm, out_hbm, didx, ones_v, zidx, acc):
    cid = lax.axis_index("c")
    sid = lax.axis_index("s")
    wid = cid * 16 + sid

    @pl.loop(0, 128)
    def _(r):
        ones_v[r, :] = jnp.zeros((16,), jnp.float32)

    base_r = sid * RSLICE
    _fill_iota_rows(zidx, base_r)

    for k in range(KSLICE):
        pltpu.sync_copy(ones_v, acc.at[zidx.at[k]])
    @pl.loop(0, 128)
    def _(r):
        ones_v[r, :] = jnp.ones((16,), jnp.float32)
    plsc.subcore_barrier()

    base = wid * RPT

    @pl.loop(0, NSUP)
    def _(s):
        pltpu.sync_copy(dst_hbm.at[pl.ds(base + s * SUP, SUP)], didx)
        for j in range(SUP):
            pltpu.sync_copy(ones_v, acc.at[didx.at[j]], add=True)

    plsc.subcore_barrier()
    for k in range(KSLICE):
        pltpu.sync_copy(acc.at[zidx.at[k]], ones_v)
        pltpu.sync_copy(
            ones_v, out_hbm.at[cid].at[pl.ds(base_r + k * 128, 128)])


@functools.partial(
    pl.kernel,
    out_type=jax.ShapeDtypeStruct((2, NP, D), jnp.float32),
    mesh=_mesh,
    scratch_types=[
        pltpu.VMEM((SUP, 128), jnp.int32),
        pltpu.VMEM((SUP, 128), jnp.int32),
        pltpu.VMEM((KSLICE, 128), jnp.int32),
        pltpu.VMEM((2, 128, D), jnp.float32),
        pltpu.VMEM_SHARED((NP, D), jnp.float32),
        pltpu.SemaphoreType.DMA,
        pltpu.SemaphoreType.DMA,
    ],
)
def _msg_kernel(hs_hbm, src_hbm, dst_hbm, out_hbm, sidx, didx, zidx, rows, acc,
                sem0, sem1):
    cid = lax.axis_index("c")
    sid = lax.axis_index("s")
    wid = cid * 16 + sid
    sems = (sem0, sem1)

    @pl.loop(0, 128)
    def _(r):
        @pl.loop(0, D // 16)
        def _(k):
            rows[0, r, pl.ds(k * 16, 16)] = jnp.zeros((16,), jnp.float32)

    base_r = sid * RSLICE
    _fill_iota_rows(zidx, base_r)
    for k in range(KSLICE):
        pltpu.sync_copy(rows.at[0], acc.at[zidx.at[k]])
    plsc.subcore_barrier()

    base = wid * RPT

    @pl.loop(0, NSUP)
    def _(s):
        pltpu.sync_copy(src_hbm.at[pl.ds(base + s * SUP, SUP)], sidx)
        pltpu.sync_copy(dst_hbm.at[pl.ds(base + s * SUP, SUP)], didx)
        pltpu.make_async_copy(hs_hbm.at[sidx.at[0]], rows.at[0], sems[0]).start()
        for j in range(SUP):
            slot = j % 2
            if j + 1 < SUP:
                nslot = (j + 1) % 2
                pltpu.make_async_copy(hs_hbm.at[sidx.at[j + 1]],
                                      rows.at[nslot], sems[nslot]).start()
            pltpu.make_async_copy(hs_hbm.at[sidx.at[j]],
                                  rows.at[slot], sems[slot]).wait()
            pltpu.sync_copy(rows.at[slot], acc.at[didx.at[j]], add=True)

    plsc.subcore_barrier()
    for k in range(KSLICE):
        pltpu.sync_copy(acc.at[zidx.at[k]], rows.at[0])
        pltpu.sync_copy(
            rows.at[0], out_hbm.at[cid].at[pl.ds(base_r + k * 128, 128)])


def _mm_body(x_ref, w_ref, b_ref, o_ref):
    o_ref[...] = (jnp.dot(x_ref[...], w_ref[...],
                          preferred_element_type=jnp.float32) + b_ref[...])


_mm1 = pl.pallas_call(
    _mm_body,
    grid=(NP // BR,),
    in_specs=[pl.BlockSpec((BR, D), lambda i: (i, 0)),
              pl.BlockSpec((D, D), lambda i: (0, 0)),
              pl.BlockSpec((1, D), lambda i: (0, 0))],
    out_specs=pl.BlockSpec((BR, D), lambda i: (i, 0)),
    out_shape=jax.ShapeDtypeStruct((NP, D), jnp.float32),
)


def _scale_body(dp_ref, xw_ref, dinv_ref, hs_ref):
    deg16 = 1.0 + dp_ref[0] + dp_ref[1]
    dinv16 = lax.rsqrt(deg16)
    dinv_ref[...] = dinv16
    hs_ref[...] = xw_ref[...] * dinv16[:, 0:1]


_scale = pl.pallas_call(
    _scale_body,
    grid=(NP // BR,),
    in_specs=[pl.BlockSpec((2, BR, 16), lambda i: (0, i, 0)),
              pl.BlockSpec((BR, D), lambda i: (i, 0))],
    out_specs=[pl.BlockSpec((BR, 16), lambda i: (i, 0)),
               pl.BlockSpec((BR, D), lambda i: (i, 0))],
    out_shape=[jax.ShapeDtypeStruct((NP, 16), jnp.float32),
               jax.ShapeDtypeStruct((NP, D), jnp.float32)],
)


def _mid_body(p_ref, hs1_ref, dinv_ref, w_ref, b_ref, o_ref):
    dinv = dinv_ref[...][:, 0:1]
    h = jnp.maximum(dinv * (p_ref[0] + p_ref[1] + hs1_ref[...]), 0.0)
    o_ref[...] = (jnp.dot(h, w_ref[...],
                          preferred_element_type=jnp.float32) + b_ref[...]) * dinv


_mid = pl.pallas_call(
    _mid_body,
    grid=(NP // BR,),
    in_specs=[pl.BlockSpec((2, BR, D), lambda i: (0, i, 0)),
              pl.BlockSpec((BR, D), lambda i: (i, 0)),
              pl.BlockSpec((BR, 16), lambda i: (i, 0)),
              pl.BlockSpec((D, D), lambda i: (0, 0)),
              pl.BlockSpec((1, D), lambda i: (0, 0))],
    out_specs=pl.BlockSpec((BR, D), lambda i: (i, 0)),
    out_shape=jax.ShapeDtypeStruct((NP, D), jnp.float32),
)


def _final_body(p_ref, hs2_ref, dinv_ref, o_ref):
    dinv = dinv_ref[...][:, 0:1]
    o_ref[...] = dinv * (p_ref[0] + p_ref[1] + hs2_ref[...])


_final = pl.pallas_call(
    _final_body,
    grid=(NP // BR,),
    in_specs=[pl.BlockSpec((2, BR, D), lambda i: (0, i, 0)),
              pl.BlockSpec((BR, D), lambda i: (i, 0)),
              pl.BlockSpec((BR, 16), lambda i: (i, 0))],
    out_specs=pl.BlockSpec((BR, D), lambda i: (i, 0)),
    out_shape=jax.ShapeDtypeStruct((NP, D), jnp.float32),
)


@jax.jit
def _run(x, edge_index, W1, b1, W2, b2):
    src = edge_index[0]
    dst = edge_index[1]
    pad_e = N + (jnp.arange(EP - E, dtype=jnp.int32) % (NP - N))
    src_p = jnp.concatenate([src, pad_e]).reshape(EROWS, 128)
    dst_p = jnp.concatenate([dst, pad_e]).reshape(EROWS, 128)
    x_p = jnp.concatenate([x, jnp.zeros((NP - N, D), x.dtype)])
    b1r = b1.reshape(1, D)
    b2r = b2.reshape(1, D)

    degp = _deg_kernel(dst_p)
    xw1 = _mm1(x_p, W1, b1r)
    dinv16, hs1 = _scale(degp, xw1)
    p1 = _msg_kernel(hs1, src_p, dst_p)
    hs2 = _mid(p1, hs1, dinv16, W2, b2r)
    p2 = _msg_kernel(hs2, src_p, dst_p)
    out = _final(p2, hs2, dinv16)
    return out[:N]


def kernel(x, edge_index, W1, b1, W2, b2):
    return _run(x, edge_index, W1, b1, W2, b2)


# speedup vs baseline: 26.0656x; 1.0603x over previous
import functools

import jax
import jax.numpy as jnp
from jax import lax
from jax.experimental import pallas as pl
from jax.experimental.pallas import tpu as pltpu
from jax.experimental.pallas import tpu_sc as plsc

N = 10000
D = 128
E = 320000

NT = 32
NP = 10240
RSLICE = NP // 16
C = 128
RPT = 80
SUP = 8
NSUP = RPT // SUP
EP = NT * RPT * C
EROWS = EP // C

BR = 2560

_mesh = plsc.VectorSubcoreMesh(core_axis_name="c", subcore_axis_name="s")


def _fill_iota_rows(zidx, npieces, plen, base_r):
    for k in range(npieces):
        @pl.loop(0, plen // 16)
        def _(t, _k=k):
            zidx[_k, pl.ds(t * 16, 16)] = (
                lax.iota(jnp.int32, 16) + (base_r + _k * plen + t * 16))


@functools.partial(
    pl.kernel,
    out_type=jax.ShapeDtypeStruct((2, NP, 16), jnp.float32),
    mesh=_mesh,
    scratch_types=[
        pltpu.VMEM((RPT, C), jnp.int32),
        pltpu.VMEM((C, 16), jnp.float32),
        pltpu.VMEM((2, 128, 16), jnp.float32),
        pltpu.VMEM((5, 128), jnp.int32),
        pltpu.VMEM_SHARED((NP, 16), jnp.float32),
        pltpu.SemaphoreType.DMA,
        pltpu.SemaphoreType.DMA,
        pltpu.SemaphoreType.DMA,
        pltpu.SemaphoreType.DMA,
        pltpu.SemaphoreType.DMA,
    ],
)
def _deg_kernel(dst_hbm, out_hbm, didx, ones_v, rb, zidx, acc,
                isem, m0, m1, m2, m3):
    cid = lax.axis_index("c")
    sid = lax.axis_index("s")
    wid = cid * 16 + sid
    msems = (m0, m1, m2, m3)
    base = wid * RPT
    base_r = sid * RSLICE

    idx_cp = pltpu.make_async_copy(dst_hbm.at[pl.ds(base, RPT)], didx, isem)
    idx_cp.start()

    @pl.loop(0, C)
    def _(r):
        ones_v[r, :] = jnp.ones((16,), jnp.float32)

    @pl.loop(0, 128)
    def _(r):
        rb[0, r, :] = jnp.zeros((16,), jnp.float32)

    _fill_iota_rows(zidx, 5, 128, base_r)

    for k in range(5):
        pltpu.make_async_copy(rb.at[0], acc.at[zidx.at[k]], m0).start()
    for k in range(5):
        pltpu.make_async_copy(rb.at[0], acc.at[zidx.at[k]], m0).wait()

    idx_cp.wait()
    plsc.subcore_barrier()

    def ds_cp(i, r):
        return pltpu.make_async_copy(ones_v, acc.at[didx.at[i]], msems[r])

    for u in range(4):
        ds_cp(u, u).start(add=True)

    @pl.loop(0, (RPT - 4) // 4)
    def _(b):
        for u in range(4):
            i = 4 + 4 * b + u
            ds_cp(i - 4, u).wait()
            ds_cp(i, u).start(add=True)

    for u in range(4):
        ds_cp(RPT - 4 + u, u).wait()
    plsc.subcore_barrier()

    def rb_cp(k, s):
        return pltpu.make_async_copy(acc.at[zidx.at[k]], rb.at[s], msems[s])

    def wr_cp(k, s):
        return pltpu.make_async_copy(
            rb.at[s], out_hbm.at[cid].at[pl.ds(base_r + k * 128, 128)],
            msems[2 + s])

    rb_cp(0, 0).start()
    for k in range(5):
        s = k % 2
        rb_cp(k, s).wait()
        wr_cp(k, s).start()
        if k + 1 < 5:
            s2 = (k + 1) % 2
            if k >= 1:
                wr_cp(k - 1, s2).wait()
            rb_cp(k + 1, s2).start()
    wr_cp(3, 1).wait()
    wr_cp(4, 0).wait()


@functools.partial(
    pl.kernel,
    out_type=jax.ShapeDtypeStruct((2, NP, D), jnp.float32),
    mesh=_mesh,
    scratch_types=[
        pltpu.VMEM((2, SUP, C), jnp.int32),
        pltpu.VMEM((2, SUP, C), jnp.int32),
        pltpu.VMEM((5, 128), jnp.int32),
        pltpu.VMEM((2 * C, D), jnp.float32),
        pltpu.VMEM_SHARED((NP, D), jnp.float32),
        pltpu.SemaphoreType.DMA,
        pltpu.SemaphoreType.DMA,
        pltpu.SemaphoreType.DMA,
        pltpu.SemaphoreType.DMA,
        pltpu.SemaphoreType.DMA,
    ],
)
def _msg_kernel(hs_hbm, src_hbm, dst_hbm, out_hbm, sidx, didx, zidx, rows, acc,
                isem, g0, g1, s0, s1):
    cid = lax.axis_index("c")
    sid = lax.axis_index("s")
    wid = cid * 16 + sid
    gsems = (g0, g1)
    ssems = (s0, s1)
    base = wid * RPT
    base_r = sid * RSLICE

    def idx_cps(S, p):
        return (
            pltpu.make_async_copy(
                src_hbm.at[pl.ds(base + S * SUP, SUP)], sidx.at[p], isem),
            pltpu.make_async_copy(
                dst_hbm.at[pl.ds(base + S * SUP, SUP)], didx.at[p], isem),
        )

    for cp in idx_cps(0, 0):
        cp.start()
    for cp in idx_cps(1, 1):
        cp.start()

    @pl.loop(0, 128)
    def _(r):
        @pl.loop(0, D // 16)
        def _(k):
            rows[r, pl.ds(k * 16, 16)] = jnp.zeros((16,), jnp.float32)

    _fill_iota_rows(zidx, 5, 128, base_r)
    for k in range(5):
        pltpu.make_async_copy(
            rows.at[pl.ds(0, 128)], acc.at[zidx.at[k]], s0).start()
    for k in range(5):
        pltpu.make_async_copy(
            rows.at[pl.ds(0, 128)], acc.at[zidx.at[k]], s0).wait()

    for cp in idx_cps(0, 0) + idx_cps(1, 1):
        cp.wait()

    def g_cp(ip, iu, r):
        return pltpu.make_async_copy(
            hs_hbm.at[sidx.at[ip, iu]], rows.at[pl.ds(r * C, C)], gsems[r])

    def s_cp(ip, iu, r):
        return pltpu.make_async_copy(
            rows.at[pl.ds(r * C, C)], acc.at[didx.at[ip, iu]], ssems[r])

    g_cp(0, 0, 0).start()
    plsc.subcore_barrier()

    def chunk(p, u, first=False, tail=False):
        r = u % 2
        g_cp(p, u, r).wait()
        s_cp(p, u, r).start(add=True)
        if not first:
            if u >= 1:
                s_cp(p, u - 1, 1 - r).wait()
            else:
                s_cp(1 - p, SUP - 1, 1 - r).wait()
        if not tail:
            if u + 1 < SUP:
                g_cp(p, u + 1, 1 - r).start()
            else:
                g_cp(1 - p, 0, 1 - r).start()

    for u in range(SUP):
        chunk(0, u, first=(u == 0))

    @pl.loop(1, NSUP - 1)
    def _(S):
        p = lax.rem(S, 2)
        for u in range(SUP):
            if u == 1:
                for cp in idx_cps(S + 1, 1 - p):
                    cp.start()
            if u == SUP - 2:
                for cp in idx_cps(S + 1, 1 - p):
                    cp.wait()
            chunk(p, u)

    for u in range(SUP):
        chunk(1, u, tail=(u == SUP - 1))
    s_cp(1, SUP - 1, (RPT - 1) % 2).wait()
    plsc.subcore_barrier()

    def rb_cp(k, s):
        return pltpu.make_async_copy(
            acc.at[zidx.at[k]], rows.at[pl.ds(s * 128, 128)], gsems[s])

    def wr_cp(k, s):
        return pltpu.make_async_copy(
            rows.at[pl.ds(s * 128, 128)],
            out_hbm.at[cid].at[pl.ds(base_r + k * 128, 128)], ssems[s])

    rb_cp(0, 0).start()
    for k in range(5):
        s = k % 2
        rb_cp(k, s).wait()
        wr_cp(k, s).start()
        if k + 1 < 5:
            s2 = (k + 1) % 2
            if k >= 1:
                wr_cp(k - 1, s2).wait()
            rb_cp(k + 1, s2).start()
    wr_cp(3, 1).wait()
    wr_cp(4, 0).wait()


def _mm_body(x_ref, w_ref, b_ref, o_ref):
    o_ref[...] = (jnp.dot(x_ref[...], w_ref[...],
                          preferred_element_type=jnp.float32) + b_ref[...])


_mm1 = pl.pallas_call(
    _mm_body,
    grid=(NP // BR,),
    in_specs=[pl.BlockSpec((BR, D), lambda i: (i, 0)),
              pl.BlockSpec((D, D), lambda i: (0, 0)),
              pl.BlockSpec((1, D), lambda i: (0, 0))],
    out_specs=pl.BlockSpec((BR, D), lambda i: (i, 0)),
    out_shape=jax.ShapeDtypeStruct((NP, D), jnp.float32),
)


def _scale_body(dp_ref, xw_ref, dinv_ref, hs_ref):
    deg16 = 1.0 + dp_ref[0] + dp_ref[1]
    dinv16 = lax.rsqrt(deg16)
    dinv_ref[...] = dinv16
    hs_ref[...] = xw_ref[...] * dinv16[:, 0:1]


_scale = pl.pallas_call(
    _scale_body,
    grid=(NP // BR,),
    in_specs=[pl.BlockSpec((2, BR, 16), lambda i: (0, i, 0)),
              pl.BlockSpec((BR, D), lambda i: (i, 0))],
    out_specs=[pl.BlockSpec((BR, 16), lambda i: (i, 0)),
               pl.BlockSpec((BR, D), lambda i: (i, 0))],
    out_shape=[jax.ShapeDtypeStruct((NP, 16), jnp.float32),
               jax.ShapeDtypeStruct((NP, D), jnp.float32)],
)


def _mid_body(p_ref, hs1_ref, dinv_ref, w_ref, b_ref, o_ref):
    dinv = dinv_ref[...][:, 0:1]
    h = jnp.maximum(dinv * (p_ref[0] + p_ref[1] + hs1_ref[...]), 0.0)
    o_ref[...] = (jnp.dot(h, w_ref[...],
                          preferred_element_type=jnp.float32) + b_ref[...]) * dinv


_mid = pl.pallas_call(
    _mid_body,
    grid=(NP // BR,),
    in_specs=[pl.BlockSpec((2, BR, D), lambda i: (0, i, 0)),
              pl.BlockSpec((BR, D), lambda i: (i, 0)),
              pl.BlockSpec((BR, 16), lambda i: (i, 0)),
              pl.BlockSpec((D, D), lambda i: (0, 0)),
              pl.BlockSpec((1, D), lambda i: (0, 0))],
    out_specs=pl.BlockSpec((BR, D), lambda i: (i, 0)),
    out_shape=jax.ShapeDtypeStruct((NP, D), jnp.float32),
)


def _final_body(p_ref, hs2_ref, dinv_ref, o_ref):
    dinv = dinv_ref[...][:, 0:1]
    o_ref[...] = dinv * (p_ref[0] + p_ref[1] + hs2_ref[...])


_final = pl.pallas_call(
    _final_body,
    grid=(NP // BR,),
    in_specs=[pl.BlockSpec((2, BR, D), lambda i: (0, i, 0)),
              pl.BlockSpec((BR, D), lambda i: (i, 0)),
              pl.BlockSpec((BR, 16), lambda i: (i, 0))],
    out_specs=pl.BlockSpec((BR, D), lambda i: (i, 0)),
    out_shape=jax.ShapeDtypeStruct((NP, D), jnp.float32),
)


@jax.jit
def _run(x, edge_index, W1, b1, W2, b2):
    src = edge_index[0]
    dst = edge_index[1]
    pad_e = N + (jnp.arange(EP - E, dtype=jnp.int32) % (NP - N))
    src_p = jnp.concatenate([src, pad_e]).reshape(EROWS, C)
    dst_p = jnp.concatenate([dst, pad_e]).reshape(EROWS, C)
    x_p = jnp.concatenate([x, jnp.zeros((NP - N, D), x.dtype)])
    b1r = b1.reshape(1, D)
    b2r = b2.reshape(1, D)

    degp = _deg_kernel(dst_p)
    xw1 = _mm1(x_p, W1, b1r)
    dinv16, hs1 = _scale(degp, xw1)
    p1 = _msg_kernel(hs1, src_p, dst_p)
    hs2 = _mid(p1, hs1, dinv16, W2, b2r)
    p2 = _msg_kernel(hs2, src_p, dst_p)
    out = _final(p2, hs2, dinv16)
    return out[:N]


def kernel(x, edge_index, W1, b1, W2, b2):
    return _run(x, edge_index, W1, b1, W2, b2)


# speedup vs baseline: 26.3605x; 1.0113x over previous
import functools

import jax
import jax.numpy as jnp
from jax import lax
from jax.experimental import pallas as pl
from jax.experimental.pallas import tpu as pltpu
from jax.experimental.pallas import tpu_sc as plsc

N = 10000
D = 128
E = 320000

NT = 32
NP = 10240
RSLICE = NP // 16
C = 128
RPT = 80
SUP = 8
NSUP = RPT // SUP
EP = NT * RPT * C
EROWS = EP // C

BR = 2560

_mesh = plsc.VectorSubcoreMesh(core_axis_name="c", subcore_axis_name="s")


def _fill_iota_rows(zidx, npieces, plen, base_r):
    for k in range(npieces):
        @pl.loop(0, plen // 16)
        def _(t, _k=k):
            zidx[_k, pl.ds(t * 16, 16)] = (
                lax.iota(jnp.int32, 16) + (base_r + _k * plen + t * 16))


@functools.partial(
    pl.kernel,
    out_type=jax.ShapeDtypeStruct((2, NP, 16), jnp.float32),
    mesh=_mesh,
    scratch_types=[
        pltpu.VMEM((RPT, C), jnp.int32),
        pltpu.VMEM((C, 16), jnp.float32),
        pltpu.VMEM((2, 128, 16), jnp.float32),
        pltpu.VMEM((5, 128), jnp.int32),
        pltpu.VMEM_SHARED((NP, 16), jnp.float32),
        pltpu.SemaphoreType.DMA,
        pltpu.SemaphoreType.DMA,
        pltpu.SemaphoreType.DMA,
        pltpu.SemaphoreType.DMA,
        pltpu.SemaphoreType.DMA,
    ],
)
def _deg_kernel(dst_hbm, out_hbm, didx, ones_v, rb, zidx, acc,
                isem, m0, m1, m2, m3):
    cid = lax.axis_index("c")
    sid = lax.axis_index("s")
    wid = cid * 16 + sid
    msems = (m0, m1, m2, m3)
    base = wid * RPT
    base_r = sid * RSLICE

    idx_cp = pltpu.make_async_copy(dst_hbm.at[pl.ds(base, RPT)], didx, isem)
    idx_cp.start()

    @pl.loop(0, C)
    def _(r):
        ones_v[r, :] = jnp.ones((16,), jnp.float32)

    @pl.loop(0, 128)
    def _(r):
        rb[0, r, :] = jnp.zeros((16,), jnp.float32)

    _fill_iota_rows(zidx, 5, 128, base_r)

    for k in range(5):
        pltpu.make_async_copy(rb.at[0], acc.at[zidx.at[k]], m0).start()
    for k in range(5):
        pltpu.make_async_copy(rb.at[0], acc.at[zidx.at[k]], m0).wait()

    idx_cp.wait()
    plsc.subcore_barrier()

    def ds_cp(i, r):
        return pltpu.make_async_copy(ones_v, acc.at[didx.at[i]], msems[r])

    for u in range(4):
        ds_cp(u, u).start(add=True)

    @pl.loop(0, (RPT - 4) // 4)
    def _(b):
        for u in range(4):
            i = 4 + 4 * b + u
            ds_cp(i - 4, u).wait()
            ds_cp(i, u).start(add=True)

    for u in range(4):
        ds_cp(RPT - 4 + u, u).wait()
    plsc.subcore_barrier()

    def rb_cp(k, s):
        return pltpu.make_async_copy(acc.at[zidx.at[k]], rb.at[s], msems[s])

    def wr_cp(k, s):
        return pltpu.make_async_copy(
            rb.at[s], out_hbm.at[cid].at[pl.ds(base_r + k * 128, 128)],
            msems[2 + s])

    rb_cp(0, 0).start()
    for k in range(5):
        s = k % 2
        rb_cp(k, s).wait()
        wr_cp(k, s).start()
        if k + 1 < 5:
            s2 = (k + 1) % 2
            if k >= 1:
                wr_cp(k - 1, s2).wait()
            rb_cp(k + 1, s2).start()
    wr_cp(3, 1).wait()
    wr_cp(4, 0).wait()


@functools.partial(
    pl.kernel,
    out_type=jax.ShapeDtypeStruct((2, NP, D), jnp.float32),
    mesh=_mesh,
    scratch_types=[
        pltpu.VMEM((2, SUP, C), jnp.int32),
        pltpu.VMEM((2, SUP, C), jnp.int32),
        pltpu.VMEM((5, 128), jnp.int32),
        pltpu.VMEM((2 * C, D), jnp.float32),
        pltpu.VMEM_SHARED((NP, D), jnp.float32),
        pltpu.SemaphoreType.DMA,
        pltpu.SemaphoreType.DMA,
        pltpu.SemaphoreType.DMA,
        pltpu.SemaphoreType.DMA,
        pltpu.SemaphoreType.DMA,
    ],
)
def _msg_kernel(hs_hbm, src_hbm, dst_hbm, out_hbm, sidx, didx, zidx, rows, acc,
                isem, g0, g1, s0, s1):
    cid = lax.axis_index("c")
    sid = lax.axis_index("s")
    wid = cid * 16 + sid
    gsems = (g0, g1)
    ssems = (s0, s1)
    base = wid * RPT
    base_r = sid * RSLICE

    def idx_cps(S, p):
        return (
            pltpu.make_async_copy(
                src_hbm.at[pl.ds(base + S * SUP, SUP)], sidx.at[p], isem),
            pltpu.make_async_copy(
                dst_hbm.at[pl.ds(base + S * SUP, SUP)], didx.at[p], isem),
        )

    for cp in idx_cps(0, 0):
        cp.start()
    for cp in idx_cps(1, 1):
        cp.start()

    @pl.loop(0, 128)
    def _(r):
        @pl.loop(0, D // 16)
        def _(k):
            rows[r, pl.ds(k * 16, 16)] = jnp.zeros((16,), jnp.float32)

    _fill_iota_rows(zidx, 5, 128, base_r)
    for k in range(5):
        pltpu.make_async_copy(
            rows.at[pl.ds(0, 128)], acc.at[zidx.at[k]], s0).start()
    for k in range(5):
        pltpu.make_async_copy(
            rows.at[pl.ds(0, 128)], acc.at[zidx.at[k]], s0).wait()

    for cp in idx_cps(0, 0) + idx_cps(1, 1):
        cp.wait()

    def g_cp(ip, iu, r):
        return pltpu.make_async_copy(
            hs_hbm.at[sidx.at[ip, iu]], rows.at[pl.ds(r * C, C)], gsems[r])

    def s_cp(ip, iu, r):
        return pltpu.make_async_copy(
            rows.at[pl.ds(r * C, C)], acc.at[didx.at[ip, iu]], ssems[r])

    g_cp(0, 0, 0).start()
    plsc.subcore_barrier()

    def chunk(p, u, first=False, tail=False):
        r = u % 2
        g_cp(p, u, r).wait()
        s_cp(p, u, r).start(add=True)
        if not first:
            if u >= 1:
                s_cp(p, u - 1, 1 - r).wait()
            else:
                s_cp(1 - p, SUP - 1, 1 - r).wait()
        if not tail:
            if u + 1 < SUP:
                g_cp(p, u + 1, 1 - r).start()
            else:
                g_cp(1 - p, 0, 1 - r).start()

    for u in range(SUP):
        chunk(0, u, first=(u == 0))

    @pl.loop(1, NSUP - 1)
    def _(S):
        p = lax.rem(S, 2)
        for u in range(SUP):
            if u == 1:
                for cp in idx_cps(S + 1, 1 - p):
                    cp.start()
            if u == SUP - 2:
                for cp in idx_cps(S + 1, 1 - p):
                    cp.wait()
            chunk(p, u)

    for u in range(SUP):
        chunk(1, u, tail=(u == SUP - 1))
    s_cp(1, SUP - 1, (RPT - 1) % 2).wait()
    plsc.subcore_barrier()

    def rb_cp(k, s):
        return pltpu.make_async_copy(
            acc.at[zidx.at[k]], rows.at[pl.ds(s * 128, 128)], gsems[s])

    def wr_cp(k, s):
        return pltpu.make_async_copy(
            rows.at[pl.ds(s * 128, 128)],
            out_hbm.at[cid].at[pl.ds(base_r + k * 128, 128)], ssems[s])

    rb_cp(0, 0).start()
    for k in range(5):
        s = k % 2
        rb_cp(k, s).wait()
        wr_cp(k, s).start()
        if k + 1 < 5:
            s2 = (k + 1) % 2
            if k >= 1:
                wr_cp(k - 1, s2).wait()
            rb_cp(k + 1, s2).start()
    wr_cp(3, 1).wait()
    wr_cp(4, 0).wait()


def _scale_body(dp_ref, x_ref, w_ref, b_ref, dinv_ref, hs_ref):
    deg16 = 1.0 + dp_ref[0] + dp_ref[1]
    dinv16 = lax.rsqrt(deg16)
    dinv_ref[...] = dinv16
    xw = (jnp.dot(x_ref[...], w_ref[...],
                  preferred_element_type=jnp.float32) + b_ref[...])
    hs_ref[...] = xw * dinv16[:, 0:1]


_scale = pl.pallas_call(
    _scale_body,
    grid=(NP // BR,),
    in_specs=[pl.BlockSpec((2, BR, 16), lambda i: (0, i, 0)),
              pl.BlockSpec((BR, D), lambda i: (i, 0)),
              pl.BlockSpec((D, D), lambda i: (0, 0)),
              pl.BlockSpec((1, D), lambda i: (0, 0))],
    out_specs=[pl.BlockSpec((BR, 16), lambda i: (i, 0)),
               pl.BlockSpec((BR, D), lambda i: (i, 0))],
    out_shape=[jax.ShapeDtypeStruct((NP, 16), jnp.float32),
               jax.ShapeDtypeStruct((NP, D), jnp.float32)],
)


def _mid_body(p_ref, hs1_ref, dinv_ref, w_ref, b_ref, o_ref):
    dinv = dinv_ref[...][:, 0:1]
    h = jnp.maximum(dinv * (p_ref[0] + p_ref[1] + hs1_ref[...]), 0.0)
    o_ref[...] = (jnp.dot(h, w_ref[...],
                          preferred_element_type=jnp.float32) + b_ref[...]) * dinv


_mid = pl.pallas_call(
    _mid_body,
    grid=(NP // BR,),
    in_specs=[pl.BlockSpec((2, BR, D), lambda i: (0, i, 0)),
              pl.BlockSpec((BR, D), lambda i: (i, 0)),
              pl.BlockSpec((BR, 16), lambda i: (i, 0)),
              pl.BlockSpec((D, D), lambda i: (0, 0)),
              pl.BlockSpec((1, D), lambda i: (0, 0))],
    out_specs=pl.BlockSpec((BR, D), lambda i: (i, 0)),
    out_shape=jax.ShapeDtypeStruct((NP, D), jnp.float32),
)


def _final_body(p_ref, hs2_ref, dinv_ref, o_ref):
    dinv = dinv_ref[...][:, 0:1]
    o_ref[...] = dinv * (p_ref[0] + p_ref[1] + hs2_ref[...])


_final = pl.pallas_call(
    _final_body,
    grid=(N // 2000,),
    in_specs=[pl.BlockSpec((2, 2000, D), lambda i: (0, i, 0)),
              pl.BlockSpec((2000, D), lambda i: (i, 0)),
              pl.BlockSpec((2000, 16), lambda i: (i, 0))],
    out_specs=pl.BlockSpec((2000, D), lambda i: (i, 0)),
    out_shape=jax.ShapeDtypeStruct((N, D), jnp.float32),
)


@jax.jit
def _run(x, edge_index, W1, b1, W2, b2):
    src = edge_index[0]
    dst = edge_index[1]
    pad_e = N + (jnp.arange(EP - E, dtype=jnp.int32) % (NP - N))
    src_p = jnp.concatenate([src, pad_e]).reshape(EROWS, C)
    dst_p = jnp.concatenate([dst, pad_e]).reshape(EROWS, C)
    x_p = jnp.concatenate([x, jnp.zeros((NP - N, D), x.dtype)])
    b1r = b1.reshape(1, D)
    b2r = b2.reshape(1, D)

    degp = _deg_kernel(dst_p)
    dinv16, hs1 = _scale(degp, x_p, W1, b1r)
    p1 = _msg_kernel(hs1, src_p, dst_p)
    hs2 = _mid(p1, hs1, dinv16, W2, b2r)
    p2 = _msg_kernel(hs2, src_p, dst_p)
    return _final(p2, hs2, dinv16)


def kernel(x, edge_index, W1, b1, W2, b2):
    return _run(x, edge_index, W1, b1, W2, b2)
